# baseline (speedup 1.0000x reference)
import jax
import jax.numpy as jnp
from jax.experimental import pallas as pl
from jax.experimental.pallas import tpu as pltpu

_NUM_GROUPS = 16
_FEAT = 128
_WIN = 32
_SPAN = 64
_OUT_ROWS = _NUM_GROUPS * _NUM_GROUPS


def _gmm_kernel(ind_hbm, gl_ref, right_ref, out_ref, ind_ref, m_ref, isem):
    icp = pltpu.make_async_copy(ind_hbm, ind_ref, isem)
    icp.start()
    for i in range(_NUM_GROUPS):
        m_ref[i] = jnp.dot(gl_ref[...], right_ref[i],
                           preferred_element_type=jnp.float32)
    icp.wait()
    for i in range(_NUM_GROUPS):
        cnt = 2 * i + 1
        start = jnp.minimum(jnp.maximum(ind_ref[i, 0], 0), _SPAN - _WIN)
        res = m_ref[i, pl.ds(start, _WIN), :]
        out_ref[i * i:i * i + cnt, :] = res[:cnt, :]


def kernel(grouped_left, right, ind_group):
    return pl.pallas_call(
        _gmm_kernel,
        grid=(1,),
        in_specs=[
            pl.BlockSpec(memory_space=pl.ANY),
            pl.BlockSpec((_SPAN, _FEAT), lambda i: (0, 0),
                         memory_space=pltpu.VMEM),
            pl.BlockSpec((_NUM_GROUPS, _FEAT, _FEAT), lambda i: (0, 0, 0),
                         memory_space=pltpu.VMEM),
        ],
        out_specs=pl.BlockSpec((_OUT_ROWS, _FEAT), lambda i: (0, 0),
                               memory_space=pltpu.VMEM),
        out_shape=jax.ShapeDtypeStruct((_OUT_ROWS, _FEAT), jnp.float32),
        scratch_shapes=[
            pltpu.SMEM((_NUM_GROUPS, 2), jnp.int32),
            pltpu.VMEM((_NUM_GROUPS, _SPAN, _FEAT), jnp.float32),
            pltpu.SemaphoreType.DMA,
        ],
    )(ind_group.astype(jnp.int32), grouped_left, right)
